# Initial kernel scaffold; baseline (speedup 1.0000x reference)
import jax
import jax.numpy as jnp
from jax.experimental import pallas as pl


def kernel(x, edge_index, W1, b1, W2, b2, W3, b3):
    raise NotImplementedError("write your pallas kernel here")


# speedup vs baseline: 3.4756x; 3.4756x over previous
import functools

import jax
import jax.numpy as jnp
from jax import lax
from jax.experimental import pallas as pl
from jax.experimental.pallas import tpu as pltpu
from jax.experimental.pallas import tpu_sc as plsc

N = 10000
D = 128
E = 320000
C = 64
NC, NS = 2, 16
NW = NC * NS
RING = 3
EROWS = 5120
TROWS = EROWS // NW
EPAD = EROWS * C - E
NPAD = 10240
RPT = NPAD // NS
ZR = 64

_MESH = plsc.VectorSubcoreMesh(
    core_axis_name="c", subcore_axis_name="s", num_cores=NC, num_subcores=NS
)


@functools.partial(
    pl.kernel,
    out_type=jax.ShapeDtypeStruct((NC, NPAD, D), jnp.float32),
    mesh=_MESH,
    scratch_types=[
        pltpu.VMEM((TROWS, C), jnp.int32),
        pltpu.VMEM((RING, C), jnp.int32),
        pltpu.VMEM((RING, C), jnp.int32),
        pltpu.VMEM((RING, C, D), jnp.float32),
        pltpu.VMEM_SHARED((NPAD, D), jnp.float32),
        pltpu.SemaphoreType.DMA,
    ],
)
def _sc_scatter(table, pk2d, zblk, out, pidx, sidx, didx, rows, acc, gsem):
    c = lax.axis_index("c")
    s = lax.axis_index("s")
    w = s * NC + c
    row0 = w * TROWS
    base = s * RPT

    pltpu.sync_copy(zblk, rows.at[0])
    for k in range(RPT // ZR):
        pltpu.sync_copy(rows.at[0], acc.at[pl.ds(base + k * ZR, ZR)])
    plsc.subcore_barrier()

    pltpu.sync_copy(pk2d.at[pl.ds(row0, TROWS)], pidx)

    def unpack(j, slot):
        def upk(g, _):
            v = pidx[j, pl.ds(g * 16, 16)]
            sidx[slot, pl.ds(g * 16, 16)] = v & 0xFFFF
            didx[slot, pl.ds(g * 16, 16)] = v >> 16
            return 0

        lax.fori_loop(0, C // 16, upk, 0)

    for i in range(RING):
        unpack(i, i)
        pltpu.async_copy(table.at[sidx.at[i]], rows.at[i], gsem)

    def body(i, _):
        slot = lax.rem(i, RING)
        pltpu.make_async_copy(table.at[sidx.at[slot]], rows.at[slot], gsem).wait()
        pltpu.sync_copy(rows.at[slot], acc.at[didx.at[slot]], add=True)

        @pl.when(i + RING < TROWS)
        def _():
            unpack(i + RING, slot)
            pltpu.async_copy(table.at[sidx.at[slot]], rows.at[slot], gsem)

        return 0

    lax.fori_loop(0, TROWS, body, 0)
    plsc.subcore_barrier()

    for k in range(RPT // ZR):
        pltpu.sync_copy(acc.at[pl.ds(base + k * ZR, ZR)], rows.at[0])
        pltpu.sync_copy(rows.at[0], out.at[c, pl.ds(base + k * ZR, ZR)])


_BM = 1000


def _mm_body(x_ref, w_ref, o_ref):
    o_ref[...] = jnp.dot(x_ref[...], w_ref[...], preferred_element_type=jnp.float32)


def _mm(x, w):
    return pl.pallas_call(
        _mm_body,
        grid=(N // _BM,),
        in_specs=[
            pl.BlockSpec((_BM, D), lambda i: (i, 0)),
            pl.BlockSpec((D, D), lambda i: (0, 0)),
        ],
        out_specs=pl.BlockSpec((_BM, D), lambda i: (i, 0)),
        out_shape=jax.ShapeDtypeStruct((N, D), jnp.float32),
    )(x, w)


def _fused_body(p_ref, b_ref, w_ref, o_ref):
    a = jnp.maximum(p_ref[0] + p_ref[1] + b_ref[...], 0.0)
    o_ref[...] = jnp.dot(a, w_ref[...], preferred_element_type=jnp.float32)


def _mm_fused(p, b, w):
    return pl.pallas_call(
        _fused_body,
        grid=(N // _BM,),
        in_specs=[
            pl.BlockSpec((NC, _BM, D), lambda i: (0, i, 0)),
            pl.BlockSpec((1, D), lambda i: (0, 0)),
            pl.BlockSpec((D, D), lambda i: (0, 0)),
        ],
        out_specs=pl.BlockSpec((_BM, D), lambda i: (i, 0)),
        out_shape=jax.ShapeDtypeStruct((N, D), jnp.float32),
    )(p, b, w)


def _final_body(p_ref, b_ref, o_ref):
    o_ref[...] = p_ref[0] + p_ref[1] + b_ref[...]


def _final(p, b):
    return pl.pallas_call(
        _final_body,
        grid=(N // _BM,),
        in_specs=[
            pl.BlockSpec((NC, _BM, D), lambda i: (0, i, 0)),
            pl.BlockSpec((1, D), lambda i: (0, 0)),
        ],
        out_specs=pl.BlockSpec((_BM, D), lambda i: (i, 0)),
        out_shape=jax.ShapeDtypeStruct((N, D), jnp.float32),
    )(p, b)


def kernel(x, edge_index, W1, b1, W2, b2, W3, b3):
    src_pad = jnp.concatenate([edge_index[0], jnp.zeros((EPAD,), jnp.int32)])
    dst_pad = jnp.concatenate(
        [edge_index[1], N + (jnp.arange(EPAD, dtype=jnp.int32) % (NPAD - N))]
    )
    pk2d = (src_pad | (dst_pad << 16)).reshape(EROWS, C)
    zblk = jnp.zeros((ZR, D), jnp.float32)
    b1r, b2r, b3r = b1.reshape(1, D), b2.reshape(1, D), b3.reshape(1, D)

    t1 = _mm(x, W1)
    p1 = _sc_scatter(t1, pk2d, zblk)
    t2 = _mm_fused(p1, b1r, W2)
    p2 = _sc_scatter(t2, pk2d, zblk)
    t3 = _mm_fused(p2, b2r, W3)
    p3 = _sc_scatter(t3, pk2d, zblk)
    return _final(p3, b3r)
